# Initial kernel scaffold; baseline (speedup 1.0000x reference)
import functools

import jax
import jax.numpy as jnp
from jax import lax
from jax.experimental import pallas as pl
from jax.experimental.pallas import tpu as pltpu
from jax.experimental.pallas import tpu_sc as plsc

N_CAMS = 16384
N_VIEWS = 256


def _transforms_body(emb_ref, t_ref):
    e = emb_ref[:]
    dx0 = e[:, 0:1]
    dx1 = e[:, 1:2]
    dx2 = e[:, 2:3]
    a1x = e[:, 3:4] + 1.0
    a1y = e[:, 4:5]
    a1z = e[:, 5:6]
    a2x = e[:, 6:7]
    a2y = e[:, 7:8] + 1.0
    a2z = e[:, 8:9]
    n1 = jnp.maximum(jnp.sqrt(a1x * a1x + a1y * a1y + a1z * a1z), 1e-12)
    b1x = a1x / n1
    b1y = a1y / n1
    b1z = a1z / n1
    d = b1x * a2x + b1y * a2y + b1z * a2z
    c2x = a2x - d * b1x
    c2y = a2y - d * b1y
    c2z = a2z - d * b1z
    n2 = jnp.maximum(jnp.sqrt(c2x * c2x + c2y * c2y + c2z * c2z), 1e-12)
    b2x = c2x / n2
    b2y = c2y / n2
    b2z = c2z / n2
    b3x = b1y * b2z - b1z * b2y
    b3y = b1z * b2x - b1x * b2z
    b3z = b1x * b2y - b1y * b2x
    zero = jnp.zeros_like(dx0)
    one = jnp.ones_like(dx0)
    t_ref[:] = jnp.concatenate(
        [b1x, b1y, b1z, dx0,
         b2x, b2y, b2z, dx1,
         b3x, b3y, b3z, dx2,
         zero, zero, zero, one],
        axis=1,
    )


def _transforms(embeds):
    return pl.pallas_call(
        _transforms_body,
        out_shape=jax.ShapeDtypeStruct((N_VIEWS, 16), jnp.float32),
    )(embeds)


_SC_INFO = plsc.get_sparse_core_info()
_NC = _SC_INFO.num_cores
_NS = _SC_INFO.num_subcores
_NW = _NC * _NS
_ROWS_PER_W = N_CAMS // _NW

_sc_mesh = plsc.VectorSubcoreMesh(core_axis_name="c", subcore_axis_name="s")


@functools.partial(
    pl.kernel,
    mesh=_sc_mesh,
    out_type=jax.ShapeDtypeStruct((N_CAMS, 16), jnp.float32),
    scratch_types=[
        pltpu.VMEM((_ROWS_PER_W,), jnp.int32),
        pltpu.VMEM((_ROWS_PER_W, 16), jnp.float32),
        pltpu.SemaphoreType.DMA,
    ],
)
def _gather_rows(table_hbm, idx_hbm, out_hbm, idx_v, rows_v, sem):
    wid = lax.axis_index("s") * _NC + lax.axis_index("c")
    base = wid * _ROWS_PER_W
    pltpu.sync_copy(idx_hbm.at[pl.ds(base, _ROWS_PER_W)], idx_v)
    pltpu.async_copy(table_hbm.at[idx_v], rows_v, sem).wait()
    pltpu.sync_copy(rows_v, out_hbm.at[pl.ds(base, _ROWS_PER_W)])


_M = N_CAMS * 16 // 128
_MB = 256


def _apply_body(a_ref, g_ref, o_ref):
    a = a_ref[:]
    g = g_ref[:]
    row = lax.broadcasted_iota(jnp.int32, (128, 128), 0)
    col = lax.broadcasted_iota(jnp.int32, (128, 128), 1)
    acc = jnp.zeros_like(a)
    for k in range(4):
        pa = (row == ((col & -4) | k)).astype(jnp.float32)
        pg = (row == ((col & -16) | (4 * k) | (col & 3))).astype(jnp.float32)
        ak = jnp.dot(a, pa, preferred_element_type=jnp.float32)
        gk = jnp.dot(g, pg, preferred_element_type=jnp.float32)
        acc = acc + ak * gk
    o_ref[:] = acc


def _apply(a, g):
    grid = (_M // _MB,)
    spec = pl.BlockSpec((_MB, 128), lambda i: (i, 0))
    return pl.pallas_call(
        _apply_body,
        grid=grid,
        in_specs=[spec, spec],
        out_specs=spec,
        out_shape=jax.ShapeDtypeStruct((_M, 128), jnp.float32),
    )(a, g)


def kernel(camtoworlds, view_ids, embeds):
    t16 = _transforms(embeds)
    g = _gather_rows(t16, view_ids.astype(jnp.int32))
    a = camtoworlds.reshape(_M, 128)
    out = _apply(a, g.reshape(_M, 128))
    return out.reshape(N_CAMS, 4, 4)


# speedup vs baseline: 1.4125x; 1.4125x over previous
import functools

import jax
import jax.numpy as jnp
from jax import lax
from jax.experimental import pallas as pl
from jax.experimental.pallas import tpu as pltpu
from jax.experimental.pallas import tpu_sc as plsc

N_CAMS = 16384
N_VIEWS = 256


def _transforms_body(emb_ref, t_ref):
    e = emb_ref[:]
    dx0 = e[:, 0:1]
    dx1 = e[:, 1:2]
    dx2 = e[:, 2:3]
    a1x = e[:, 3:4] + 1.0
    a1y = e[:, 4:5]
    a1z = e[:, 5:6]
    a2x = e[:, 6:7]
    a2y = e[:, 7:8] + 1.0
    a2z = e[:, 8:9]
    n1 = jnp.maximum(jnp.sqrt(a1x * a1x + a1y * a1y + a1z * a1z), 1e-12)
    b1x = a1x / n1
    b1y = a1y / n1
    b1z = a1z / n1
    d = b1x * a2x + b1y * a2y + b1z * a2z
    c2x = a2x - d * b1x
    c2y = a2y - d * b1y
    c2z = a2z - d * b1z
    n2 = jnp.maximum(jnp.sqrt(c2x * c2x + c2y * c2y + c2z * c2z), 1e-12)
    b2x = c2x / n2
    b2y = c2y / n2
    b2z = c2z / n2
    b3x = b1y * b2z - b1z * b2y
    b3y = b1z * b2x - b1x * b2z
    b3z = b1x * b2y - b1y * b2x
    zero = jnp.zeros_like(dx0)
    one = jnp.ones_like(dx0)
    t_ref[:] = jnp.concatenate(
        [b1x, b1y, b1z, dx0,
         b2x, b2y, b2z, dx1,
         b3x, b3y, b3z, dx2,
         zero, zero, zero, one],
        axis=1,
    )


def _transforms(embeds):
    return pl.pallas_call(
        _transforms_body,
        out_shape=jax.ShapeDtypeStruct((N_VIEWS, 16), jnp.float32),
    )(embeds)


_NC = 2
_NS = 16
_NW = _NC * _NS
_ROWS_PER_W = N_CAMS // _NW

@functools.cache
def _make_gather_rows():
    mesh = plsc.VectorSubcoreMesh(
        core_axis_name="c", subcore_axis_name="s", num_cores=_NC
    )

    @functools.partial(
        pl.kernel,
        mesh=mesh,
        out_type=jax.ShapeDtypeStruct((N_CAMS, 16), jnp.float32),
        scratch_types=[
            pltpu.VMEM((_ROWS_PER_W,), jnp.int32),
            pltpu.VMEM((_ROWS_PER_W, 16), jnp.float32),
            pltpu.SemaphoreType.DMA,
        ],
        compiler_params=pltpu.CompilerParams(use_tc_tiling_on_sc=False),
    )
    def _gather_rows(table_hbm, idx_hbm, out_hbm, idx_v, rows_v, sem):
        wid = lax.axis_index("s") * _NC + lax.axis_index("c")
        base = wid * _ROWS_PER_W
        pltpu.sync_copy(idx_hbm.at[pl.ds(base, _ROWS_PER_W)], idx_v)
        pltpu.async_copy(table_hbm.at[idx_v], rows_v, sem).wait()
        pltpu.sync_copy(rows_v, out_hbm.at[pl.ds(base, _ROWS_PER_W)])

    return _gather_rows


_M = N_CAMS * 16 // 128
_MB = 256


def _apply_body(a_ref, g_ref, o_ref):
    a = a_ref[:]
    g = g_ref[:]
    row = lax.broadcasted_iota(jnp.int32, (128, 128), 0)
    col = lax.broadcasted_iota(jnp.int32, (128, 128), 1)
    acc = jnp.zeros_like(a)
    for k in range(4):
        pa = (row == ((col & -4) | k)).astype(jnp.float32)
        pg = (row == ((col & -16) | (4 * k) | (col & 3))).astype(jnp.float32)
        ak = jnp.dot(a, pa, preferred_element_type=jnp.float32)
        gk = jnp.dot(g, pg, preferred_element_type=jnp.float32)
        acc = acc + ak * gk
    o_ref[:] = acc


def _apply(a, g):
    grid = (_M // _MB,)
    spec = pl.BlockSpec((_MB, 128), lambda i: (i, 0))
    return pl.pallas_call(
        _apply_body,
        grid=grid,
        in_specs=[spec, spec],
        out_specs=spec,
        out_shape=jax.ShapeDtypeStruct((_M, 128), jnp.float32),
    )(a, g)


def kernel(camtoworlds, view_ids, embeds):
    t16 = _transforms(embeds)
    g = _make_gather_rows()(t16, view_ids.astype(jnp.int32))
    a = camtoworlds.reshape(_M, 128)
    out = _apply(a, g.reshape(_M, 128))
    return out.reshape(N_CAMS, 4, 4)


# speedup vs baseline: 8.3746x; 5.9287x over previous
import functools

import jax
import jax.numpy as jnp
from jax import lax
from jax.experimental import pallas as pl
from jax.experimental.pallas import tpu as pltpu
from jax.experimental.pallas import tpu_sc as plsc

N_CAMS = 16384
N_VIEWS = 256
_NC = 2
_NS = 16
_NW = _NC * _NS
_CPW = N_CAMS // _NW
_L = 16
_GROUPS = _CPW // _L


def _rsqrt16(s):
    i = lax.bitcast_convert_type(s, jnp.int32)
    i = jnp.int32(0x5F3759DF) - lax.shift_right_arithmetic(i, 1)
    y = lax.bitcast_convert_type(i, jnp.float32)
    half_s = 0.5 * s
    for _ in range(3):
        y = y * (1.5 - half_s * y * y)
    return jnp.minimum(y, jnp.float32(1e12))


@functools.cache
def _make_sc_kernel():
    mesh = plsc.VectorSubcoreMesh(
        core_axis_name="c", subcore_axis_name="s", num_cores=_NC
    )

    @functools.partial(
        pl.kernel,
        mesh=mesh,
        out_type=jax.ShapeDtypeStruct((4, N_CAMS * 4), jnp.float32),
        scratch_types=[
            pltpu.VMEM((9, N_VIEWS), jnp.float32),
            pltpu.VMEM((16, N_VIEWS), jnp.float32),
            pltpu.VMEM((_CPW,), jnp.int32),
            pltpu.VMEM((4 * _CPW * 4,), jnp.float32),
            pltpu.VMEM((4 * _CPW * 4,), jnp.float32),
        ],
        compiler_params=pltpu.CompilerParams(
            use_tc_tiling_on_sc=False, needs_layout_passes=False
        ),
    )
    def _sc_kernel(emb_hbm, vid_hbm, cam_hbm, out_hbm,
                   emb_v, tab_v, vid_v, cam_v, out_v):
        wid = lax.axis_index("s") * _NC + lax.axis_index("c")
        row_len = _CPW * 4

        pltpu.sync_copy(emb_hbm, emb_v)
        pltpu.sync_copy(vid_hbm.at[pl.ds(wid * _CPW, _CPW)], vid_v)
        for r in range(4):
            pltpu.sync_copy(
                cam_hbm.at[r, pl.ds(wid * row_len, row_len)],
                cam_v.at[pl.ds(r * row_len, row_len)],
            )

        one = jnp.ones((_L,), jnp.float32)
        zero = jnp.zeros((_L,), jnp.float32)
        for vt in range(N_VIEWS // _L):
            sl = pl.ds(vt * _L, _L)
            dx0 = emb_v[0, sl]
            dx1 = emb_v[1, sl]
            dx2 = emb_v[2, sl]
            a1x = emb_v[3, sl] + 1.0
            a1y = emb_v[4, sl]
            a1z = emb_v[5, sl]
            a2x = emb_v[6, sl]
            a2y = emb_v[7, sl] + 1.0
            a2z = emb_v[8, sl]
            inv1 = _rsqrt16(a1x * a1x + a1y * a1y + a1z * a1z)
            b1x = a1x * inv1
            b1y = a1y * inv1
            b1z = a1z * inv1
            d = b1x * a2x + b1y * a2y + b1z * a2z
            c2x = a2x - d * b1x
            c2y = a2y - d * b1y
            c2z = a2z - d * b1z
            inv2 = _rsqrt16(c2x * c2x + c2y * c2y + c2z * c2z)
            b2x = c2x * inv2
            b2y = c2y * inv2
            b2z = c2z * inv2
            b3x = b1y * b2z - b1z * b2y
            b3y = b1z * b2x - b1x * b2z
            b3z = b1x * b2y - b1y * b2x
            comps = (b1x, b1y, b1z, dx0,
                     b2x, b2y, b2z, dx1,
                     b3x, b3y, b3z, dx2,
                     zero, zero, zero, one)
            for j, v in enumerate(comps):
                tab_v[j, sl] = v

        jsplat = [jnp.full((_L,), j, jnp.int32) for j in range(16)]

        def body(g, carry):
            ti = g // 8
            sub = g % 8
            vids = vid_v[pl.ds(g * _L, _L)]
            gcomp = [
                plsc.load_gather(tab_v, [jsplat[j], vids]) for j in range(16)
            ]
            lane0 = ti * 512 + sub * _L
            for r in range(4):
                rbase = r * row_len + lane0
                a = [cam_v[pl.ds(rbase + k * 128, _L)] for k in range(4)]
                for c in range(4):
                    acc = a[0] * gcomp[c]
                    acc = acc + a[1] * gcomp[4 + c]
                    acc = acc + a[2] * gcomp[8 + c]
                    acc = acc + a[3] * gcomp[12 + c]
                    out_v[pl.ds(rbase + c * 128, _L)] = acc
            return carry

        lax.fori_loop(0, _GROUPS, body, 0)

        for r in range(4):
            pltpu.sync_copy(
                out_v.at[pl.ds(r * row_len, row_len)],
                out_hbm.at[r, pl.ds(wid * row_len, row_len)],
            )

    return _sc_kernel


def kernel(camtoworlds, view_ids, embeds):
    cam_lin = jnp.transpose(
        camtoworlds.reshape(128, 128, 4, 4), (2, 0, 3, 1)
    ).reshape(4, N_CAMS * 4)
    out_lin = _make_sc_kernel()(
        embeds.T, view_ids.astype(jnp.int32), cam_lin
    )
    return jnp.transpose(
        out_lin.reshape(4, 128, 4, 128), (1, 3, 0, 2)
    ).reshape(N_CAMS, 4, 4)
